# baseline (speedup 1.0000x reference)
import functools

import jax
import jax.numpy as jnp
from jax import lax
from jax.experimental import pallas as pl
from jax.experimental.pallas import tpu as pltpu
from jax.experimental.pallas import tpu_sc as plsc

NC = 2
NS = 16
NW = NC * NS
BT = 128
D = 64
L = 16
RS = 129


@functools.partial(jax.jit, static_argnums=(2,))
def _embed_t(table2, tokT, B0):
    T = tokT.shape[0]
    n_bt = B0 // BT
    n_items = T * n_bt
    per_w = n_items // NW

    mesh = plsc.VectorSubcoreMesh(
        core_axis_name="c", subcore_axis_name="s",
        num_cores=NC, num_subcores=NS)

    @functools.partial(
        pl.kernel,
        out_type=jax.ShapeDtypeStruct((T, D // 8, n_bt, 8, BT), jnp.float32),
        mesh=mesh,
        scratch_types=[
            pltpu.VMEM((2, BT), jnp.int32),
            pltpu.VMEM((2, BT), jnp.int32),
            pltpu.VMEM((2, BT), jnp.int32),
            pltpu.VMEM((2, BT, RS), jnp.float32),
            pltpu.VMEM((2, D, BT), jnp.float32),
            pltpu.SemaphoreType.DMA,
            pltpu.SemaphoreType.DMA,
            pltpu.SemaphoreType.DMA,
            pltpu.SemaphoreType.DMA,
            pltpu.SemaphoreType.DMA,
            pltpu.SemaphoreType.DMA,
        ],
        compiler_params=pltpu.CompilerParams(use_tc_tiling_on_sc=True,
                                             needs_layout_passes=False),
    )
    def k(table_hbm, tok_hbm, ot_hbm, tok_v, q_v, hb_v, rows_v, tr_v,
          t0, t1, g0, g1, w0, w1):
        tsems = (t0, t1)
        gsems = (g0, g1)
        wsems = (w0, w1)
        wid = lax.axis_index("s") * NC + lax.axis_index("c")
        n0 = wid * per_w
        iota = lax.iota(jnp.int32, L)
        ivecs = [iota + g * L for g in range(8)]

        def t_bt(n):
            return n >> 7, n & (n_bt - 1)

        def tok_load(n, b):
            t, bt = t_bt(n)
            pltpu.async_copy(tok_hbm.at[t, pl.ds(bt * BT, BT)],
                             tok_v.at[b], tsems[b])

        def stage_b(n, b):
            t, bt = t_bt(n)
            pltpu.make_async_copy(tok_hbm.at[t, pl.ds(bt * BT, BT)],
                                  tok_v.at[b], tsems[b]).wait()
            for g in range(8):
                tok_g = tok_v[b, pl.ds(g * L, L)]
                q_v[b, pl.ds(g * L, L)] = lax.shift_right_logical(tok_g, 1)
                hb_v[b, pl.ds(g * L, L)] = lax.shift_left(tok_g & 1, 6)
            pltpu.async_copy(table_hbm.at[q_v.at[b]],
                             rows_v.at[b, :, pl.ds(0, 2 * D)], gsems[b])

        def stage_c(n, b):
            t, bt = t_bt(n)
            pltpu.make_async_copy(table_hbm.at[q_v.at[b]],
                                  rows_v.at[b, :, pl.ds(0, 2 * D)],
                                  gsems[b]).wait()

            @pl.when(n - n0 >= 2)
            def _():
                tp, btp = t_bt(n - 2)
                for ct in range(D // 8):
                    pltpu.make_async_copy(tr_v.at[b, pl.ds(ct * 8, 8)],
                                          ot_hbm.at[tp, ct, btp],
                                          wsems[b]).wait()

            cols = [hb_v[b, pl.ds(g * L, L)] for g in range(8)]

            @plsc.parallel_loop(0, D, unroll=8)
            def _tp(c):
                for g in range(8):
                    vals = plsc.load_gather(rows_v.at[b],
                                            [ivecs[g], cols[g] + c])
                    tr_v[b, c, pl.ds(g * L, L)] = vals

            for ct in range(D // 8):
                pltpu.async_copy(tr_v.at[b, pl.ds(ct * 8, 8)],
                                 ot_hbm.at[t, ct, bt], wsems[b])

        tok_load(n0, 0)
        tok_load(n0 + 1, 1)

        @pl.loop(n0, n0 + per_w, step=2)
        def _blk(jo):
            for b in (0, 1):
                n = jo + b
                stage_b(n, b)

                @pl.when(n - n0 >= 1)
                def _():
                    stage_c(n - 1, 1 - b)

                @pl.when(n - n0 < per_w - 2)
                def _():
                    tok_load(n + 2, b)

        stage_c(n0 + per_w - 1, 1)
        for b, back in ((0, 2), (1, 1)):
            t, bt = t_bt(n0 + per_w - back)
            for ct in range(D // 8):
                pltpu.make_async_copy(tr_v.at[b, pl.ds(ct * 8, 8)],
                                      ot_hbm.at[t, ct, bt], wsems[b]).wait()

    return k(table2, tokT)


def kernel(tokens, W_E):
    B0, T = tokens.shape
    V, _D = W_E.shape
    table2 = W_E.reshape(V // 2, 2 * _D)
    tokT = tokens.T
    ot = _embed_t(table2, tokT, B0)
    return ot.transpose(2, 4, 0, 1, 3).reshape(B0, T, _D)


# speedup vs baseline: 1.6688x; 1.6688x over previous
import functools

import jax
import jax.numpy as jnp
from jax import lax
from jax.experimental import pallas as pl
from jax.experimental.pallas import tpu as pltpu
from jax.experimental.pallas import tpu_sc as plsc

NC = 2
NS = 16
NW = NC * NS
C = 128
G = 2
S = C * G


@functools.partial(jax.jit, static_argnums=(2, 3))
def _gather_rows(table, idx2d, B, D):
    n_chunks = idx2d.shape[0]
    b_per_w = B // NW
    n_slots = b_per_w // S
    chunks_per_w = n_chunks // NW

    mesh = plsc.VectorSubcoreMesh(
        core_axis_name="c", subcore_axis_name="s",
        num_cores=NC, num_subcores=NS)

    @functools.partial(
        pl.kernel,
        out_type=jax.ShapeDtypeStruct((B, 2 * D), jnp.float32),
        mesh=mesh,
        scratch_types=[
            pltpu.VMEM((2, G, C), jnp.int32),
            pltpu.VMEM((2, S, 2 * D), jnp.float32),
            pltpu.SemaphoreType.DMA,
            pltpu.SemaphoreType.DMA,
            pltpu.SemaphoreType.DMA,
            pltpu.SemaphoreType.DMA,
        ],
        compiler_params=pltpu.CompilerParams(use_tc_tiling_on_sc=True,
                                             needs_layout_passes=False),
    )
    def k(table_hbm, idx_hbm, out_hbm, idx_v, rows_v, g0, g1, s0, s1):
        gsems = (g0, g1)
        ssems = (s0, s1)
        wid = lax.axis_index("s") * NC + lax.axis_index("c")
        base = wid * b_per_w
        chunk_base = wid * chunks_per_w

        def idx_load(j, b):
            pltpu.sync_copy(idx_hbm.at[pl.ds(chunk_base + j * G, G)],
                            idx_v.at[b])

        def fire_gathers(j, b):
            for g in range(G):
                pltpu.async_copy(table_hbm.at[idx_v.at[b, g]],
                                 rows_v.at[b, pl.ds(g * C, C)], gsems[b])

        def wait_gathers(j, b):
            for g in range(G):
                pltpu.make_async_copy(table_hbm.at[idx_v.at[b, g]],
                                      rows_v.at[b, pl.ds(g * C, C)],
                                      gsems[b]).wait()

        def fire_scatter(j, b):
            pltpu.async_copy(rows_v.at[b],
                             out_hbm.at[pl.ds(base + j * S, S)], ssems[b])

        def wait_scatter(j, b):
            pltpu.make_async_copy(rows_v.at[b],
                                  out_hbm.at[pl.ds(base + j * S, S)],
                                  ssems[b]).wait()

        idx_load(0, 0)
        fire_gathers(0, 0)

        @pl.loop(0, n_slots, step=2)
        def _slot(jo):
            for b in (0, 1):
                j = jo + b

                @pl.when(j + 1 < n_slots)
                def _():
                    idx_load(j + 1, 1 - b)

                    @pl.when(j >= 1)
                    def _():
                        wait_scatter(j - 1, 1 - b)

                    fire_gathers(j + 1, 1 - b)

                wait_gathers(j, b)
                fire_scatter(j, b)

        wait_scatter(n_slots - 2, 0)
        wait_scatter(n_slots - 1, 1)

    return k(table, idx2d)


def kernel(tokens, W_E):
    B0, T = tokens.shape
    V, D = W_E.shape
    B = B0 * T
    table_p = jnp.pad(W_E, ((0, 0), (0, 128 - D)))
    idx2d = tokens.reshape(B // C, C)
    out = _gather_rows(table_p, idx2d, B, D)
    return out.reshape(B0, T, 2 * D)[:, :, :D]
